# Initial kernel scaffold; baseline (speedup 1.0000x reference)
import jax
import jax.numpy as jnp
from jax.experimental import pallas as pl


def kernel(pts, time, plane0, plane1, plane2):
    raise NotImplementedError("write your pallas kernel here")


# speedup vs baseline: 52.8086x; 52.8086x over previous
import functools

import jax
import jax.numpy as jnp
import numpy as np
from jax import lax
from jax.experimental import pallas as pl
from jax.experimental.pallas import tpu as pltpu
from jax.experimental.pallas import tpu_sc as plsc

NN = 524288
FEAT = 32
H, W = 128, 64
NROWS = H * W

NC, NS, L = 2, 16, 16
NWORK = NC * NS
CH = 1024
GB = 128
NB = CH // GB
CHUNKS = NN // (NWORK * CH)
PER_W = NN // NWORK

SCALE = float(np.float32(2.0) / (np.float32(-1.3) - np.float32(1.3)))
TS = float(np.float32(1.0 / 128.0))


def _indices(pv, tv):
    x = (tv + 1.0) * 0.5 * float(W - 1)
    ix = jnp.minimum(jnp.maximum((x + 0.5).astype(jnp.int32), 0), W - 1)
    p = (pv - 1.3) * SCALE - 1.0
    y = (p + 1.0) * 0.5 * float(H - 1)
    iy = jnp.minimum(jnp.maximum((y + 0.5).astype(jnp.int32), 0), H - 1)
    return iy * W + ix


def _sc_sample(ptsT, tvec, tab0, tab1, tab2, cond16):
    mesh = plsc.VectorSubcoreMesh(core_axis_name="c", subcore_axis_name="s")
    out_t = jax.ShapeDtypeStruct((NN, FEAT), jnp.float32)

    @functools.partial(
        pl.kernel,
        out_type=[out_t, out_t],
        mesh=mesh,
        compiler_params=pltpu.CompilerParams(use_tc_tiling_on_sc=False),
        scratch_types=[
            pltpu.VMEM((3, CH), jnp.float32),
            pltpu.VMEM((CH,), jnp.float32),
            pltpu.VMEM((NB, GB), jnp.int32),
            pltpu.VMEM((NB, GB), jnp.int32),
            pltpu.VMEM((NB, GB), jnp.int32),
            pltpu.VMEM((CH, FEAT), jnp.float32),
            pltpu.VMEM((CH, FEAT), jnp.float32),
            pltpu.VMEM((CH, FEAT), jnp.float32),
            pltpu.VMEM((3, L), jnp.float32),
            pltpu.VMEM((L,), jnp.float32),
            pltpu.VMEM((3, L), jnp.int32),
            pltpu.VMEM((3, L, FEAT), jnp.float32),
            pltpu.VMEM((FEAT,), jnp.float32),
            pltpu.VMEM((FEAT,), jnp.float32),
            pltpu.VMEM((L,), jnp.int32),
            pltpu.SemaphoreType.DMA,
        ],
    )
    def body(ptsT_h, tvec_h, t0_h, t1_h, t2_h, c16_h, outA, outB,
             cbuf, tbuf, ib0, ib1, ib2, rb0, rb1, rb2,
             pbuf, ptbuf, fibuf, frbuf, fabuf, fbbuf, cvbuf, sem):
        wid = lax.axis_index("s") * NC + lax.axis_index("c")
        wbase = wid * PER_W

        def chunk_body(c, _):
            base = wbase + c * CH
            pltpu.sync_copy(ptsT_h.at[:, pl.ds(base, CH)], cbuf)
            pltpu.sync_copy(tvec_h.at[pl.ds(base, CH)], tbuf)

            def idx_block(b, _):
                def idx_group(u, _):
                    o = b * GB + u * L
                    col = u * L
                    tv = tbuf[pl.ds(o, L)] * 2.0 - 1.0
                    ib0[b, pl.ds(col, L)] = _indices(cbuf[0, pl.ds(o, L)], tv)
                    ib1[b, pl.ds(col, L)] = _indices(cbuf[1, pl.ds(o, L)], tv)
                    ib2[b, pl.ds(col, L)] = _indices(cbuf[2, pl.ds(o, L)], tv)
                    return 0
                return lax.fori_loop(0, GB // L, idx_group, 0)
            lax.fori_loop(0, NB, idx_block, 0)

            copies = []
            for b in range(NB):
                dst = pl.ds(b * GB, GB)
                copies.append(pltpu.async_copy(t0_h.at[ib0.at[b]], rb0.at[dst], sem))
                copies.append(pltpu.async_copy(t1_h.at[ib1.at[b]], rb1.at[dst], sem))
                copies.append(pltpu.async_copy(t2_h.at[ib2.at[b]], rb2.at[dst], sem))
            for cp in copies:
                cp.wait()

            def mul_row(j, _):
                for h in (0, L):
                    s = pl.ds(h, L)
                    rb0[j, s] = (rb0[j, s] * rb1[j, s]) * rb2[j, s]
                return 0
            lax.fori_loop(0, CH, mul_row, 0)

            pltpu.sync_copy(rb0, outA.at[pl.ds(base, CH)])
            pltpu.sync_copy(rb0, outB.at[pl.ds(base, CH)])
            return 0

        lax.fori_loop(0, CHUNKS, chunk_body, 0)

        @pl.when(wid == NWORK - 1)
        def _fixup():
            pltpu.sync_copy(ptsT_h.at[:, pl.ds(NN - L, L)], pbuf)
            pltpu.sync_copy(tvec_h.at[pl.ds(NN - L, L)], ptbuf)
            pltpu.sync_copy(c16_h, cvbuf)
            tv = (ptbuf[...] * 2.0 - 1.0) - TS
            x = (tv + 1.0) * 0.5 * float(W - 1)
            ix = jnp.minimum(jnp.maximum((x + 0.5).astype(jnp.int32), 0), W - 1)
            for c in range(3):
                p = ((pbuf[c, :] - 1.3) * SCALE - 1.0) - TS
                y = (p + 1.0) * 0.5 * float(H - 1)
                iy = jnp.minimum(jnp.maximum((y + 0.5).astype(jnp.int32), 0), H - 1)
                fibuf[c, :] = iy * W + ix
            pltpu.async_copy(t0_h.at[fibuf.at[0]], frbuf.at[0], sem).wait()
            pltpu.async_copy(t1_h.at[fibuf.at[1]], frbuf.at[1], sem).wait()
            pltpu.async_copy(t2_h.at[fibuf.at[2]], frbuf.at[2], sem).wait()
            cv = cvbuf[...] != 0
            for h in (0, L):
                s = pl.ds(h, L)
                sh = (frbuf[0, L - 1, s] * frbuf[1, L - 1, s]) * frbuf[2, L - 1, s]
                u = rb0[CH - 1, s]
                fabuf[s] = jnp.where(cv, sh, u)
                fbbuf[s] = jnp.where(cv, u, sh)
            pltpu.sync_copy(fabuf, outA.at[NN - 1])
            pltpu.sync_copy(fbbuf, outB.at[NN - 1])

    return body(ptsT, tvec, tab0, tab1, tab2, cond16)


def kernel(pts, time, plane0, plane1, plane2):
    ptsT = pts.T
    tvec = time[:, 0]
    tab0 = jnp.transpose(plane0, (1, 2, 0)).reshape(NROWS, FEAT)
    tab1 = jnp.transpose(plane1, (1, 2, 0)).reshape(NROWS, FEAT)
    tab2 = jnp.transpose(plane2, (1, 2, 0)).reshape(NROWS, FEAT)
    px_last = (pts[-1, 0] - np.float32(1.3)) * np.float32(SCALE) - np.float32(1.0)
    cond16 = jnp.broadcast_to((px_last + TS > 1.0).astype(jnp.int32), (L,))
    feature_a, feature_b = _sc_sample(ptsT, tvec, tab0, tab1, tab2, cond16)
    return (feature_a, feature_b)
